# Initial kernel scaffold; baseline (speedup 1.0000x reference)
import jax
import jax.numpy as jnp
from jax.experimental import pallas as pl


def kernel(x, Emb, W1, b1, W2, b2, Wf, bf):
    raise NotImplementedError("write your pallas kernel here")


# speedup vs baseline: 4.9451x; 4.9451x over previous
import functools

import jax
import jax.numpy as jnp
from jax import lax
from jax.experimental import pallas as pl
from jax.experimental.pallas import tpu as pltpu
from jax.experimental.pallas import tpu_sc as plsc

_BATCH = 16384
_FIELDS = 100
_FPAD = 104
_DIM = 64
_NC = 2
_NS = 16
_NW = _NC * _NS
_BPW = _BATCH // _NW


def _bip_sc(x_pad, emb):
  mesh = plsc.VectorSubcoreMesh(core_axis_name="c", subcore_axis_name="s")

  @functools.partial(
      pl.kernel,
      out_type=jax.ShapeDtypeStruct((_BATCH, _DIM), jnp.float32),
      mesh=mesh,
      scratch_types=[
          pltpu.VMEM((_BPW, _FPAD), jnp.int32),
          pltpu.VMEM((_FPAD, _DIM), jnp.float32),
          pltpu.VMEM((_FPAD, _DIM), jnp.float32),
          pltpu.VMEM((_BPW, _DIM), jnp.float32),
          pltpu.SemaphoreType.DMA,
          pltpu.SemaphoreType.DMA,
      ],
      compiler_params=pltpu.CompilerParams(use_tc_tiling_on_sc=False),
  )
  def k(x_hbm, emb_hbm, h_hbm, idx_v, rows_a, rows_b, out_v, sem_a, sem_b):
    wid = lax.axis_index("s") * _NC + lax.axis_index("c")
    base = wid * _BPW
    pltpu.sync_copy(x_hbm.at[pl.ds(base, _BPW)], idx_v)

    def start(i, rows, sem):
      pltpu.make_async_copy(emb_hbm.at[idx_v.at[i]], rows, sem).start()

    def wait(rows, sem):
      pltpu.make_async_copy(emb_hbm.at[idx_v.at[0]], rows, sem).wait()

    def process(i, rows):
      zero = jnp.zeros((16,), jnp.float32)

      def body(f, carry):
        s0, s1, s2, s3, q0, q1, q2, q3 = carry
        v0 = rows[f, pl.ds(0, 16)]
        v1 = rows[f, pl.ds(16, 16)]
        v2 = rows[f, pl.ds(32, 16)]
        v3 = rows[f, pl.ds(48, 16)]
        return (s0 + v0, s1 + v1, s2 + v2, s3 + v3,
                q0 + v0 * v0, q1 + v1 * v1, q2 + v2 * v2, q3 + v3 * v3)

      acc = lax.fori_loop(0, _FIELDS, body, (zero,) * 8, unroll=4)
      for c in range(4):
        s, q = acc[c], acc[4 + c]
        out_v[i, pl.ds(c * 16, 16)] = (s * s - q) * 0.5

    start(0, rows_a, sem_a)
    start(1, rows_b, sem_b)

    def step(j, carry):
      i0 = 2 * j
      wait(rows_a, sem_a)
      process(i0, rows_a)
      start(i0 + 2, rows_a, sem_a)
      wait(rows_b, sem_b)
      process(i0 + 1, rows_b)
      start(i0 + 3, rows_b, sem_b)
      return carry

    lax.fori_loop(0, _BPW // 2 - 1, step, 0)
    wait(rows_a, sem_a)
    process(_BPW - 2, rows_a)
    wait(rows_b, sem_b)
    process(_BPW - 1, rows_b)
    pltpu.sync_copy(out_v, h_hbm.at[pl.ds(base, _BPW)])

  return k(x_pad, emb)


def _mlp_tc(h, w1t, b1, w2t, b2, wf, bf):
  blk = 1024

  def body(h_ref, w1_ref, b1_ref, w2_ref, b2_ref, wf_ref, bf_ref, o_ref):
    hb = h_ref[...]
    a1 = jnp.maximum(
        jnp.dot(hb, w1_ref[...], preferred_element_type=jnp.float32)
        + b1_ref[...], 0.0)
    a2 = jnp.maximum(
        jnp.dot(a1, w2_ref[...], preferred_element_type=jnp.float32)
        + b2_ref[...], 0.0)
    t = jnp.sum(a2 * wf_ref[...], axis=1, keepdims=True) + bf_ref[...]
    o_ref[...] = 1.0 / (1.0 + jnp.exp(-t))

  return pl.pallas_call(
      body,
      grid=(_BATCH // blk,),
      in_specs=[
          pl.BlockSpec((blk, _DIM), lambda i: (i, 0)),
          pl.BlockSpec((_DIM, 32), lambda i: (0, 0)),
          pl.BlockSpec((1, 32), lambda i: (0, 0)),
          pl.BlockSpec((32, 16), lambda i: (0, 0)),
          pl.BlockSpec((1, 16), lambda i: (0, 0)),
          pl.BlockSpec((1, 16), lambda i: (0, 0)),
          pl.BlockSpec((1, 1), lambda i: (0, 0)),
      ],
      out_specs=pl.BlockSpec((blk, 1), lambda i: (i, 0)),
      out_shape=jax.ShapeDtypeStruct((_BATCH, 1), jnp.float32),
  )(h, w1t, b1, w2t, b2, wf, bf)


def kernel(x, Emb, W1, b1, W2, b2, Wf, bf):
  x = x.astype(jnp.int32)
  xp = jnp.pad(x, ((0, 0), (0, _FPAD - _FIELDS)))
  h = _bip_sc(xp, Emb)
  return _mlp_tc(h, W1.T, b1.reshape(1, -1), W2.T, b2.reshape(1, -1),
                 Wf, bf.reshape(1, 1))
